# baseline (speedup 1.0000x reference)
import functools

import jax
import jax.numpy as jnp
from jax import lax
from jax.experimental import pallas as pl
from jax.experimental.pallas import tpu as pltpu
from jax.experimental.pallas import tpu_sc as plsc

_VOCAB = 1000000
_D = 64
_B = 4096
_H = 200
_TOTAL = _B * _H
_NW = 32
_PER_W = _TOTAL // _NW
_C = 128
_NCHUNK = _PER_W // _C

_mesh = plsc.VectorSubcoreMesh(core_axis_name="c", subcore_axis_name="s")


@functools.partial(
    pl.kernel,
    out_type=jax.ShapeDtypeStruct((_TOTAL, _D), jnp.float32),
    mesh=_mesh,
    scratch_types=[
        pltpu.VMEM((_C,), jnp.int32),
        pltpu.VMEM((_C, _D), jnp.float32),
        pltpu.SemaphoreType.DMA,
    ],
    compiler_params=pltpu.CompilerParams(use_tc_tiling_on_sc=False),
)
def _gather(idx_hbm, table_hbm, out_hbm, idx_v, rows_v, sem):
    wid = lax.axis_index("s") * 2 + lax.axis_index("c")
    wbase = wid * _PER_W

    def body(i, carry):
        base = wbase + i * _C
        pltpu.sync_copy(idx_hbm.at[pl.ds(base, _C)], idx_v)
        pltpu.async_copy(table_hbm.at[idx_v], rows_v, sem).wait()
        pltpu.sync_copy(rows_v, out_hbm.at[pl.ds(base, _C)])
        return carry

    lax.fori_loop(0, _NCHUNK, body, 0)


def kernel(indices, table):
    flat = indices.reshape(-1)
    out = _gather(flat, table)
    return out.reshape(_B, _H, _D)


# speedup vs baseline: 1.1914x; 1.1914x over previous
import functools

import jax
import jax.numpy as jnp
from jax import lax
from jax.experimental import pallas as pl
from jax.experimental.pallas import tpu as pltpu
from jax.experimental.pallas import tpu_sc as plsc

_VOCAB = 1000000
_D = 64
_B = 4096
_H = 200
_TOTAL = _B * _H
_NW = 32
_PER_W = _TOTAL // _NW
_C = 256
_NCHUNK = _PER_W // _C
_NB = 4
_K = 2

_mesh = plsc.VectorSubcoreMesh(core_axis_name="c", subcore_axis_name="s")


@functools.partial(
    pl.kernel,
    out_type=jax.ShapeDtypeStruct((_TOTAL, _D), jnp.float32),
    mesh=_mesh,
    scratch_types=[
        pltpu.VMEM((_PER_W,), jnp.int32),
        [pltpu.VMEM((_C, _D), jnp.float32) for _ in range(_NB)],
        [pltpu.SemaphoreType.DMA for _ in range(_NB)],
        [pltpu.SemaphoreType.DMA for _ in range(_NB)],
    ],
    compiler_params=pltpu.CompilerParams(use_tc_tiling_on_sc=False),
)
def _gather(idx_hbm, table_hbm, out_hbm, idx_v, rows, gsem, wsem):
    wid = lax.axis_index("s") * 2 + lax.axis_index("c")
    wbase = wid * _PER_W

    pltpu.sync_copy(idx_hbm.at[pl.ds(wbase, _PER_W)], idx_v)

    def start_gather(chunk, b):
        pltpu.async_copy(
            table_hbm.at[idx_v.at[pl.ds(chunk * _C, _C)]], rows[b], gsem[b]
        )

    def out_slice(chunk):
        return out_hbm.at[pl.ds(wbase + chunk * _C, _C)]

    for t in range(_K):
        start_gather(t, t)

    @pl.loop(0, _NCHUNK, step=_NB)
    def turn(t0):
        for b in range(_NB):
            t = t0 + b
            pa = (b + _K) % _NB

            @pl.when(t + _K < _NCHUNK)
            def _issue_ahead():
                @pl.when(t + _K >= _NB)
                def _drain_prev_writeback():
                    pltpu.make_async_copy(
                        rows[pa], out_slice(t + _K - _NB), wsem[pa]
                    ).wait()

                start_gather(t + _K, pa)

            pltpu.make_async_copy(
                table_hbm.at[idx_v.at[pl.ds(t * _C, _C)]], rows[b], gsem[b]
            ).wait()
            pltpu.async_copy(rows[b], out_slice(t), wsem[b])

    for b in range(_NB):
        pltpu.make_async_copy(
            rows[b], out_slice(_NCHUNK - _NB + b), wsem[b]
        ).wait()


def kernel(indices, table):
    flat = indices.reshape(-1)
    out = _gather(flat, table)
    return out.reshape(_B, _H, _D)
